# baseline (speedup 1.0000x reference)
import functools

import jax
import jax.numpy as jnp
from jax import lax
from jax.experimental import pallas as pl
from jax.experimental.pallas import tpu as pltpu
from jax.experimental.pallas import tpu_sc as plsc

N = 50000
FIN = 9
E = 800000
H = 64
G = 128
C = 10
EMD = 200
DEPTH = 2

ROWS = 400
NTILES = N // ROWS

NC_SC = 2
NS_SC = 16
HW = H // 2
KCH = 256
AGGR = 50176
WR = AGGR // NS_SC
EPAD = NS_SC * AGGR
EWP = EPAD // NS_SC
NCHW = EWP // KCH
SINK = N + 16


def _scatter_round(hL, hR, srcp, dstp, zrows, out2, agg_sh, c, s,
                   sidx, didx, rb, semI, semG, semS):
    pltpu.sync_copy(zrows, agg_sh.at[pl.ds(s * WR, WR)])
    plsc.subcore_barrier()

    def run(h):
        def issue_idx(t, p):
            base = s * EWP + t * KCH
            pltpu.async_copy(srcp.at[pl.ds(base, KCH)], sidx[p], semI[p])
            pltpu.async_copy(dstp.at[pl.ds(base, KCH)], didx[p], semI[p])

        def wait_idx(t, p):
            base = s * EWP + t * KCH
            pltpu.make_async_copy(srcp.at[pl.ds(base, KCH)], sidx[p], semI[p]).wait()
            pltpu.make_async_copy(dstp.at[pl.ds(base, KCH)], didx[p], semI[p]).wait()

        issue_idx(0, 0)
        issue_idx(1, 1)
        wait_idx(0, 0)
        pltpu.async_copy(h.at[sidx[0]], rb[0], semG[0])

        def body(t, carry):
            def step(p, q):
                pltpu.make_async_copy(h.at[sidx[p]], rb[p], semG[p]).wait()

                @pl.when(t + 1 < NCHW)
                def _():
                    @pl.when(t >= 1)
                    def _():
                        pltpu.make_async_copy(rb[q], agg_sh.at[didx[q]], semS[q]).wait()
                    wait_idx(t + 1, q)
                    pltpu.async_copy(h.at[sidx[q]], rb[q], semG[q])

                pltpu.async_copy(rb[p], agg_sh.at[didx[p]], semS[p], add=True)

                @pl.when(t + 2 < NCHW)
                def _():
                    issue_idx(t + 2, p)

            @pl.when(lax.rem(t, 2) == 0)
            def _():
                step(0, 1)

            @pl.when(lax.rem(t, 2) == 1)
            def _():
                step(1, 0)

            return carry

        lax.fori_loop(0, NCHW, body, jnp.int32(0))
        pltpu.make_async_copy(rb[0], agg_sh.at[didx[0]], semS[0]).wait()
        pltpu.make_async_copy(rb[1], agg_sh.at[didx[1]], semS[1]).wait()

    @pl.when(c == 0)
    def _():
        run(hL)

    @pl.when(c == 1)
    def _():
        run(hR)

    plsc.subcore_barrier()
    pltpu.sync_copy(agg_sh.at[pl.ds(s * WR, WR)], out2.at[c, pl.ds(s * WR, WR)])
    plsc.subcore_barrier()


def _sc_scatter_body(hL, hR, srcp, dstp, zrows, out, agg_sh,
                     sidx0, sidx1, didx0, didx1, rb0, rb1,
                     semI0, semI1, semG0, semG1, semS0, semS1):
    c = lax.axis_index("c")
    s = lax.axis_index("s")
    _scatter_round(hL, hR, srcp, dstp, zrows, out, agg_sh, c, s,
                   (sidx0, sidx1), (didx0, didx1), (rb0, rb1),
                   (semI0, semI1), (semG0, semG1), (semS0, semS1))


def _sc_scatter3_body(hL0, hR0, hL1, hR1, hL2, hR2, srcp, dstp, zrows, out,
                      agg_sh, sidx0, sidx1, didx0, didx1, rb0, rb1,
                      semI0, semI1, semG0, semG1, semS0, semS1):
    c = lax.axis_index("c")
    s = lax.axis_index("s")
    pairs = ((hL0, hR0), (hL1, hR1), (hL2, hR2))
    for k in range(3):
        _scatter_round(pairs[k][0], pairs[k][1], srcp, dstp, zrows,
                       out.at[k], agg_sh, c, s,
                       (sidx0, sidx1), (didx0, didx1), (rb0, rb1),
                       (semI0, semI1), (semG0, semG1), (semS0, semS1))


_SC_SCATTER_CACHE = []


def _get_sc_scatter():
    if not _SC_SCATTER_CACHE:
        mesh = plsc.VectorSubcoreMesh(core_axis_name="c", subcore_axis_name="s")
        _SC_SCATTER_CACHE.append(pl.kernel(
            _sc_scatter_body,
            out_type=jax.ShapeDtypeStruct((NC_SC, AGGR, HW), jnp.float32),
            mesh=mesh,
            scratch_types=[
                pltpu.VMEM_SHARED((AGGR, HW), jnp.float32),
                pltpu.VMEM((KCH,), jnp.int32),
                pltpu.VMEM((KCH,), jnp.int32),
                pltpu.VMEM((KCH,), jnp.int32),
                pltpu.VMEM((KCH,), jnp.int32),
                pltpu.VMEM((KCH, HW), jnp.float32),
                pltpu.VMEM((KCH, HW), jnp.float32),
                pltpu.SemaphoreType.DMA,
                pltpu.SemaphoreType.DMA,
                pltpu.SemaphoreType.DMA,
                pltpu.SemaphoreType.DMA,
                pltpu.SemaphoreType.DMA,
                pltpu.SemaphoreType.DMA,
            ],
            compiler_params=pltpu.CompilerParams(use_tc_tiling_on_sc=False),
        ))
    return _SC_SCATTER_CACHE[0]


def _scatter(hL, hR, srcp, dstp, zrows):
    o2 = _get_sc_scatter()(hL, hR, srcp, dstp, zrows)
    return o2[0, :N], o2[1, :N]


_SC_SCATTER3_CACHE = []


def _get_sc_scatter3():
    if not _SC_SCATTER3_CACHE:
        mesh = plsc.VectorSubcoreMesh(core_axis_name="c", subcore_axis_name="s")
        _SC_SCATTER3_CACHE.append(pl.kernel(
            _sc_scatter3_body,
            out_type=jax.ShapeDtypeStruct((3, NC_SC, AGGR, HW), jnp.float32),
            mesh=mesh,
            scratch_types=[
                pltpu.VMEM_SHARED((AGGR, HW), jnp.float32),
                pltpu.VMEM((KCH,), jnp.int32),
                pltpu.VMEM((KCH,), jnp.int32),
                pltpu.VMEM((KCH,), jnp.int32),
                pltpu.VMEM((KCH,), jnp.int32),
                pltpu.VMEM((KCH, HW), jnp.float32),
                pltpu.VMEM((KCH, HW), jnp.float32),
                pltpu.SemaphoreType.DMA,
                pltpu.SemaphoreType.DMA,
                pltpu.SemaphoreType.DMA,
                pltpu.SemaphoreType.DMA,
                pltpu.SemaphoreType.DMA,
                pltpu.SemaphoreType.DMA,
            ],
            compiler_params=pltpu.CompilerParams(use_tc_tiling_on_sc=False),
        ))
    return _SC_SCATTER3_CACHE[0]


def _scatter3(hs, srcp, dstp, zrows):
    o = _get_sc_scatter3()(*hs, srcp, dstp, zrows)
    return [(o[k, 0, :N], o[k, 1, :N]) for k in range(3)]


def _split_body(ei_ref, src_ref, dst_ref):
    src_ref[pl.ds(0, E)] = ei_ref[0, :]
    dst_ref[pl.ds(0, E)] = ei_ref[1, :]
    src_ref[pl.ds(E, EPAD - E)] = jnp.zeros((EPAD - E,), jnp.int32)
    dst_ref[pl.ds(E, EPAD - E)] = jnp.full((EPAD - E,), SINK, jnp.int32)


def _tc_split_edges(edge_index):
    return pl.pallas_call(
        _split_body,
        out_shape=[
            jax.ShapeDtypeStruct((EPAD,), jnp.int32),
            jax.ShapeDtypeStruct((EPAD,), jnp.int32),
        ],
    )(edge_index.astype(jnp.int32))


def _emb_body(x_ref, emb_ref, xe_ref, l_ref, r_ref, z_ref, st_ref, acc_ref):
    i = pl.program_id(0)
    f32 = jnp.float32
    iota_e = lax.broadcasted_iota(jnp.int32, (ROWS, EMD), 1)
    acc = jnp.zeros((ROWS, H), f32)
    for k in range(FIN):
        oh = (x_ref[:, k][:, None] == iota_e).astype(f32)
        acc = acc + jnp.dot(oh, emb_ref[k], preferred_element_type=f32)
    xe_ref[...] = acc
    l_ref[...] = acc[:, :HW]
    r_ref[...] = acc[:, HW:]

    @pl.when(i == 0)
    def _():
        z_ref[...] = jnp.zeros_like(z_ref)
        acc_ref[...] = jnp.zeros_like(acc_ref)

    acc_ref[0, :] += jnp.sum(acc, axis=0)
    acc_ref[1, :] += jnp.sum(acc * acc, axis=0)

    @pl.when(i == NTILES - 1)
    def _():
        st_ref[...] = acc_ref[...]


def _tc_embed(x, atom_emb):
    return pl.pallas_call(
        _emb_body,
        grid=(NTILES,),
        in_specs=[
            pl.BlockSpec((ROWS, FIN), lambda i: (i, 0)),
            pl.BlockSpec((FIN, EMD, H), lambda i: (0, 0, 0)),
        ],
        out_specs=[
            pl.BlockSpec((ROWS, H), lambda i: (i, 0)),
            pl.BlockSpec((ROWS, HW), lambda i: (i, 0)),
            pl.BlockSpec((ROWS, HW), lambda i: (i, 0)),
            pl.BlockSpec((WR, HW), lambda i: (0, 0)),
            pl.BlockSpec((2, H), lambda i: (0, 0)),
        ],
        out_shape=[
            jax.ShapeDtypeStruct((N, H), jnp.float32),
            jax.ShapeDtypeStruct((N, HW), jnp.float32),
            jax.ShapeDtypeStruct((N, HW), jnp.float32),
            jax.ShapeDtypeStruct((WR, HW), jnp.float32),
            jax.ShapeDtypeStruct((2, H), jnp.float32),
        ],
        scratch_shapes=[pltpu.VMEM((2, H), jnp.float32)],
    )(x.astype(jnp.int32), atom_emb)


def _tail_body(e0_ref, e1_ref, e2_ref, b_ref, st_ref, Wg_ref, bg_ref, Wl_ref,
               bl_ref, Wc_ref, bc_ref, Wm1_ref, bm1_ref, Wm2_ref, bm2_ref,
               out_ref, kld_ref, diff_ref, mi_ref,
               segg_ref, segl_ref, M_ref, cjb_ref, ja0_ref, sm_ref):
    i = pl.program_id(0)
    f32 = jnp.float32
    dn = (((0,), (0,)), ((), ()))

    @pl.when(i == 0)
    def _():
        segg_ref[...] = jnp.zeros_like(segg_ref)
        segl_ref[...] = jnp.zeros_like(segl_ref)
        M_ref[...] = jnp.zeros_like(M_ref)
        sm_ref[0, 0] = -1e30
        sm_ref[0, 1] = 0.0
        sm_ref[0, 2] = 0.0

    e0 = e0_ref[...]
    e1 = e1_ref[...]
    e2 = e2_ref[...]
    bids = b_ref[0, 0, :]
    iota_g = lax.broadcasted_iota(jnp.int32, (ROWS, G), 1)
    oh = (bids[:, None] == iota_g).astype(f32)
    segg_ref[...] += lax.dot_general(oh, e1, dn, preferred_element_type=f32)
    segl_ref[...] += lax.dot_general(oh, e0, dn, preferred_element_type=f32)

    n0 = jnp.sqrt(jnp.sum(e0 * e0, axis=1, keepdims=True))
    n1 = jnp.sqrt(jnp.sum(e1 * e1, axis=1, keepdims=True))
    a2 = e0 / (n0 + 1e-6)
    b2 = e1 / (n1 + 1e-6)
    M_ref[...] += lax.dot_general(a2, b2, dn, preferred_element_type=f32)

    Wm1a = Wm1_ref[:H, :]
    Wm1b = Wm1_ref[H:, :]
    ja = jnp.dot(e1, Wm1a, preferred_element_type=f32)
    jb = jnp.dot(e2, Wm1b, preferred_element_type=f32)
    bm1 = bm1_ref[0, :]
    bm2 = bm2_ref[0, 0]
    joint_h = jnp.maximum(ja + jb + bm1, 0.0)
    joint_v = jnp.dot(joint_h, Wm2_ref[...], preferred_element_type=f32)[:, 0] + bm2
    sm_ref[0, 2] += jnp.sum(joint_v)

    jb_shift = jnp.concatenate([cjb_ref[...], jb[:ROWS - 1]], axis=0)
    cjb_ref[...] = jb[ROWS - 1:ROWS]

    @pl.when(i == 0)
    def _():
        ja0_ref[...] = ja[0:1]

    marg_h = jnp.maximum(ja + jb_shift + bm1, 0.0)
    marg_v = jnp.dot(marg_h, Wm2_ref[...], preferred_element_type=f32)[:, 0] + bm2
    rowid = lax.broadcasted_iota(jnp.int32, (ROWS,), 0)
    valid = (i > 0) | (rowid > 0)
    marg_v = jnp.where(valid, marg_v, -1e30)

    mold = sm_ref[0, 0]
    mnew = jnp.maximum(mold, jnp.max(marg_v))
    sm_ref[0, 1] = sm_ref[0, 1] * jnp.exp(mold - mnew) + jnp.sum(jnp.exp(marg_v - mnew))
    sm_ref[0, 0] = mnew

    @pl.when(i == NTILES - 1)
    def _():
        mh0 = jnp.maximum(ja0_ref[...] + cjb_ref[...] + bm1, 0.0)
        mv0 = jnp.dot(mh0, Wm2_ref[...], preferred_element_type=f32)[0, 0] + bm2
        m_all = jnp.maximum(sm_ref[0, 0], mv0)
        s_all = sm_ref[0, 1] * jnp.exp(sm_ref[0, 0] - m_all) + jnp.exp(mv0 - m_all)
        lse = jnp.log(s_all) + m_all
        mi_ref[...] = jnp.broadcast_to(sm_ref[0, 2] * (1.0 / N) - (lse - jnp.log(jnp.float32(N))), (1, 1))

        xg = jnp.maximum(jnp.dot(segg_ref[...], Wg_ref[...],
                                 preferred_element_type=f32) + bg_ref[0, :], 0.0)
        xl = jnp.maximum(jnp.dot(segl_ref[...], Wl_ref[...],
                                 preferred_element_type=f32) + bl_ref[0, :], 0.0)
        out_ref[...] = jnp.dot(xl + xg, Wc_ref[...],
                               preferred_element_type=f32) + bc_ref[0, :]

        Mv = M_ref[...]
        diff_ref[...] = jnp.broadcast_to(jnp.sum(Mv * Mv) * (1.0 / (H * H)), (1, 1))

        mu = st_ref[0, :] * (1.0 / N)
        var1 = (st_ref[1, :] - N * mu * mu) * (1.0 / (N - 1))
        kld_ref[...] = jnp.broadcast_to(-0.5 * jnp.mean(1.0 + jnp.log(var1) - mu * mu - var1), (1, 1))


def _tc_tail(e0, e1, e2, batch3, xstats, Wg, bg, Wl, bl, Wclf, bclf, Wm1, bm1, Wm2, bm2):
    cm = lambda i: (0, 0)
    return pl.pallas_call(
        _tail_body,
        grid=(NTILES,),
        in_specs=[
            pl.BlockSpec((ROWS, H), lambda i: (i, 0)),
            pl.BlockSpec((ROWS, H), lambda i: (i, 0)),
            pl.BlockSpec((ROWS, H), lambda i: (i, 0)),
            pl.BlockSpec((1, 1, ROWS), lambda i: (i, 0, 0)),
            pl.BlockSpec((2, H), cm),
            pl.BlockSpec((H, H), cm),
            pl.BlockSpec((1, H), cm),
            pl.BlockSpec((H, H), cm),
            pl.BlockSpec((1, H), cm),
            pl.BlockSpec((H, C), cm),
            pl.BlockSpec((1, C), cm),
            pl.BlockSpec((2 * H, H), cm),
            pl.BlockSpec((1, H), cm),
            pl.BlockSpec((H, 1), cm),
            pl.BlockSpec((1, 1), cm),
        ],
        out_specs=[
            pl.BlockSpec((G, C), cm),
            pl.BlockSpec((1, 1), cm),
            pl.BlockSpec((1, 1), cm),
            pl.BlockSpec((1, 1), cm),
        ],
        out_shape=[
            jax.ShapeDtypeStruct((G, C), jnp.float32),
            jax.ShapeDtypeStruct((1, 1), jnp.float32),
            jax.ShapeDtypeStruct((1, 1), jnp.float32),
            jax.ShapeDtypeStruct((1, 1), jnp.float32),
        ],
        scratch_shapes=[
            pltpu.VMEM((G, H), jnp.float32),
            pltpu.VMEM((G, H), jnp.float32),
            pltpu.VMEM((H, H), jnp.float32),
            pltpu.VMEM((1, H), jnp.float32),
            pltpu.VMEM((1, H), jnp.float32),
            pltpu.SMEM((1, 4), jnp.float32),
        ],
    )(e0, e1, e2, batch3, xstats, Wg, bg, Wl, bl, Wclf, bclf, Wm1, bm1, Wm2, bm2)


def _passA_gen_body(nh, *refs):
    i = pl.program_id(0)
    f32 = jnp.float32
    if nh == 1:
        h_ref, aL_ref, aR_ref, W_ref, b_ref, s_ref, stats_ref, acc_ref = refs
        ms = [h_ref[...] + jnp.concatenate([aL_ref[...], aR_ref[...]], axis=1)] * 3
    else:
        (h0, h1, h2, aL0, aR0, aL1, aR1, aL2, aR2,
         W_ref, b_ref, s_ref, stats_ref, acc_ref) = refs
        ms = [
            h0[...] + jnp.concatenate([aL0[...], aR0[...]], axis=1),
            h1[...] + jnp.concatenate([aL1[...], aR1[...]], axis=1),
            h2[...] + jnp.concatenate([aL2[...], aR2[...]], axis=1),
        ]

    @pl.when(i == 0)
    def _():
        acc_ref[...] = jnp.zeros_like(acc_ref)

    for k in range(3):
        s = jnp.dot(ms[k], W_ref[k], preferred_element_type=f32) + b_ref[k]
        s_ref[k] = s
        acc_ref[k, 0, :] += jnp.sum(s, axis=0)
        acc_ref[k, 1, :] += jnp.sum(s * s, axis=0)

    @pl.when(i == NTILES - 1)
    def _():
        stats_ref[...] = acc_ref[...]


def _passA(h, aggL, aggR, W1, b1):
    rb = lambda i: (i, 0)
    return pl.pallas_call(
        functools.partial(_passA_gen_body, 1),
        grid=(NTILES,),
        in_specs=[
            pl.BlockSpec((ROWS, H), rb),
            pl.BlockSpec((ROWS, HW), rb),
            pl.BlockSpec((ROWS, HW), rb),
            pl.BlockSpec((3, H, H), lambda i: (0, 0, 0)),
            pl.BlockSpec((3, 1, H), lambda i: (0, 0, 0)),
        ],
        out_specs=[
            pl.BlockSpec((3, ROWS, H), lambda i: (0, i, 0)),
            pl.BlockSpec((3, 2, H), lambda i: (0, 0, 0)),
        ],
        out_shape=[
            jax.ShapeDtypeStruct((3, N, H), jnp.float32),
            jax.ShapeDtypeStruct((3, 2, H), jnp.float32),
        ],
        scratch_shapes=[pltpu.VMEM((3, 2, H), jnp.float32)],
    )(h, aggL, aggR, W1, b1)


def _passA3(hs, aggs, W1, b1):
    rb = lambda i: (i, 0)
    h0, h1, h2 = hs
    (aL0, aR0), (aL1, aR1), (aL2, aR2) = aggs
    return pl.pallas_call(
        functools.partial(_passA_gen_body, 3),
        grid=(NTILES,),
        in_specs=[
            pl.BlockSpec((ROWS, H), rb),
            pl.BlockSpec((ROWS, H), rb),
            pl.BlockSpec((ROWS, H), rb),
        ] + [pl.BlockSpec((ROWS, HW), rb)] * 6 + [
            pl.BlockSpec((3, H, H), lambda i: (0, 0, 0)),
            pl.BlockSpec((3, 1, H), lambda i: (0, 0, 0)),
        ],
        out_specs=[
            pl.BlockSpec((3, ROWS, H), lambda i: (0, i, 0)),
            pl.BlockSpec((3, 2, H), lambda i: (0, 0, 0)),
        ],
        out_shape=[
            jax.ShapeDtypeStruct((3, N, H), jnp.float32),
            jax.ShapeDtypeStruct((3, 2, H), jnp.float32),
        ],
        scratch_shapes=[pltpu.VMEM((3, 2, H), jnp.float32)],
    )(h0, h1, h2, aL0, aR0, aL1, aR1, aL2, aR2, W1, b1)


def _passB3_body(relu_out, split, s_ref, stats_ref, gamma_ref, beta_ref, W2_ref, b2_ref,
                 *out_refs):
    f32 = jnp.float32
    for k in range(3):
        mu = stats_ref[k, 0, :] * (1.0 / N)
        var = stats_ref[k, 1, :] * (1.0 / N) - mu * mu
        inv = lax.rsqrt(var + 1e-5)
        m = (s_ref[k] - mu) * (inv * gamma_ref[k, 0, :]) + beta_ref[k, 0, :]
        m = jnp.maximum(m, 0.0)
        o = jnp.dot(m, W2_ref[k], preferred_element_type=f32) + b2_ref[k, 0, :]
        if relu_out:
            o = jnp.maximum(o, 0.0)
        out_refs[k][...] = o
        if split:
            out_refs[3 + 2 * k][...] = o[:, :HW]
            out_refs[4 + 2 * k][...] = o[:, HW:]


def _passB3(s3, stats3, gamma, beta, W2, b2, relu_out, split):
    rb = lambda i: (i, 0)
    cm = lambda i: (0, 0, 0)
    out_specs = [pl.BlockSpec((ROWS, H), rb)] * 3
    out_shape = [jax.ShapeDtypeStruct((N, H), jnp.float32)] * 3
    if split:
        out_specs += [pl.BlockSpec((ROWS, HW), rb)] * 6
        out_shape += [jax.ShapeDtypeStruct((N, HW), jnp.float32)] * 6
    return pl.pallas_call(
        functools.partial(_passB3_body, relu_out, split),
        grid=(NTILES,),
        in_specs=[
            pl.BlockSpec((3, ROWS, H), lambda i: (0, i, 0)),
            pl.BlockSpec((3, 2, H), cm),
            pl.BlockSpec((3, 1, H), cm),
            pl.BlockSpec((3, 1, H), cm),
            pl.BlockSpec((3, H, H), cm),
            pl.BlockSpec((3, 1, H), cm),
        ],
        out_specs=out_specs,
        out_shape=out_shape,
    )(s3, stats3, gamma, beta, W2, b2)


def kernel(x, edge_index, batch, atom_emb, gin_W1, gin_b1, gin_gamma, gin_beta,
           gin_W2, gin_b2, Wg, bg, Wl, bl, Wclf, bclf, Wm1, bm1, Wm2, bm2):
    f32 = jnp.float32
    srcp, dstp = _tc_split_edges(edge_index)
    xe, xeL, xeR, zrows, xstats = _tc_embed(x, atom_emb)

    b1r = gin_b1.reshape(3, DEPTH, 1, H)
    b2r = gin_b2.reshape(3, DEPTH, 1, H)
    gmr = gin_gamma.reshape(3, DEPTH, 1, H)
    btr = gin_beta.reshape(3, DEPTH, 1, H)

    aL0, aR0 = _scatter(xeL, xeR, srcp, dstp, zrows)
    s_all, stats_all = _passA(xe, aL0, aR0, gin_W1[:, 0], b1r[:, 0])
    (h0, h1, h2, hL0, hR0, hL1, hR1, hL2, hR2) = _passB3(
        s_all, stats_all, gmr[:, 0], btr[:, 0], gin_W2[:, 0], b2r[:, 0],
        relu_out=True, split=True)

    aggs = _scatter3((hL0, hR0, hL1, hR1, hL2, hR2), srcp, dstp, zrows)
    s3, st3 = _passA3((h0, h1, h2), aggs, gin_W1[:, 1], b1r[:, 1])
    encs = _passB3(s3, st3, gmr[:, 1], btr[:, 1], gin_W2[:, 1], b2r[:, 1],
                   relu_out=False, split=False)

    batch3 = batch.astype(jnp.int32).reshape(NTILES, 1, ROWS)
    out, kld2, diff2, mi2 = _tc_tail(
        encs[0], encs[1], encs[2], batch3, xstats,
        Wg, bg.reshape(1, H), Wl, bl.reshape(1, H),
        Wclf, bclf.reshape(1, C), Wm1, bm1.reshape(1, H),
        Wm2, bm2.reshape(1, 1))
    return (out, kld2[0, 0], jnp.float32(0.0), diff2[0, 0], mi2[0, 0])


# speedup vs baseline: 1.0494x; 1.0494x over previous
import functools

import jax
import jax.numpy as jnp
from jax import lax
from jax.experimental import pallas as pl
from jax.experimental.pallas import tpu as pltpu
from jax.experimental.pallas import tpu_sc as plsc

N = 50000
FIN = 9
E = 800000
H = 64
G = 128
C = 10
EMD = 200
DEPTH = 2

ROWS = 400
NTILES = N // ROWS

NC_SC = 2
NS_SC = 16
HW = H // 2
KCH = 256
AGGR = 50176
WR = AGGR // NS_SC
EPAD = NS_SC * AGGR
EWP = EPAD // NS_SC
NCHW = EWP // KCH
SINK = N + 16


def _scatter_round(hL, hR, srcp, dstp, zrows, out2, agg_sh, c, s,
                   sidx, didx, rb, semI, semG, semS):
    pltpu.sync_copy(zrows, agg_sh.at[pl.ds(s * WR, WR)])
    plsc.subcore_barrier()

    def run(h):
        def issue_idx(t, p):
            base = s * EWP + t * KCH
            pltpu.async_copy(srcp.at[pl.ds(base, KCH)], sidx[p], semI[p])
            pltpu.async_copy(dstp.at[pl.ds(base, KCH)], didx[p], semI[p])

        def wait_idx(t, p):
            base = s * EWP + t * KCH
            pltpu.make_async_copy(srcp.at[pl.ds(base, KCH)], sidx[p], semI[p]).wait()
            pltpu.make_async_copy(dstp.at[pl.ds(base, KCH)], didx[p], semI[p]).wait()

        issue_idx(0, 0)
        issue_idx(1, 1)
        wait_idx(0, 0)
        pltpu.async_copy(h.at[sidx[0]], rb[0], semG[0])

        def body(t, carry):
            def step(p, q):
                pltpu.make_async_copy(h.at[sidx[p]], rb[p], semG[p]).wait()

                @pl.when(t + 1 < NCHW)
                def _():
                    @pl.when(t >= 1)
                    def _():
                        pltpu.make_async_copy(rb[q], agg_sh.at[didx[q]], semS[q]).wait()
                    wait_idx(t + 1, q)
                    pltpu.async_copy(h.at[sidx[q]], rb[q], semG[q])

                pltpu.async_copy(rb[p], agg_sh.at[didx[p]], semS[p], add=True)

                @pl.when(t + 2 < NCHW)
                def _():
                    issue_idx(t + 2, p)

            @pl.when(lax.rem(t, 2) == 0)
            def _():
                step(0, 1)

            @pl.when(lax.rem(t, 2) == 1)
            def _():
                step(1, 0)

            return carry

        lax.fori_loop(0, NCHW, body, jnp.int32(0))
        pltpu.make_async_copy(rb[0], agg_sh.at[didx[0]], semS[0]).wait()
        pltpu.make_async_copy(rb[1], agg_sh.at[didx[1]], semS[1]).wait()

    @pl.when(c == 0)
    def _():
        run(hL)

    @pl.when(c == 1)
    def _():
        run(hR)

    plsc.subcore_barrier()
    pltpu.sync_copy(agg_sh.at[pl.ds(s * WR, WR)], out2.at[c, pl.ds(s * WR, WR)])
    plsc.subcore_barrier()


def _sc_scatter_body(hL, hR, srcp, dstp, zrows, out, agg_sh,
                     sidx0, sidx1, didx0, didx1, rb0, rb1,
                     semI0, semI1, semG0, semG1, semS0, semS1):
    c = lax.axis_index("c")
    s = lax.axis_index("s")
    _scatter_round(hL, hR, srcp, dstp, zrows, out, agg_sh, c, s,
                   (sidx0, sidx1), (didx0, didx1), (rb0, rb1),
                   (semI0, semI1), (semG0, semG1), (semS0, semS1))


_SC_SCATTER_CACHE = []


def _get_sc_scatter():
    if not _SC_SCATTER_CACHE:
        mesh = plsc.VectorSubcoreMesh(core_axis_name="c", subcore_axis_name="s")
        _SC_SCATTER_CACHE.append(pl.kernel(
            _sc_scatter_body,
            out_type=jax.ShapeDtypeStruct((NC_SC, AGGR, HW), jnp.float32),
            mesh=mesh,
            scratch_types=[
                pltpu.VMEM_SHARED((AGGR, HW), jnp.float32),
                pltpu.VMEM((KCH,), jnp.int32),
                pltpu.VMEM((KCH,), jnp.int32),
                pltpu.VMEM((KCH,), jnp.int32),
                pltpu.VMEM((KCH,), jnp.int32),
                pltpu.VMEM((KCH, HW), jnp.float32),
                pltpu.VMEM((KCH, HW), jnp.float32),
                pltpu.SemaphoreType.DMA,
                pltpu.SemaphoreType.DMA,
                pltpu.SemaphoreType.DMA,
                pltpu.SemaphoreType.DMA,
                pltpu.SemaphoreType.DMA,
                pltpu.SemaphoreType.DMA,
            ],
            compiler_params=pltpu.CompilerParams(use_tc_tiling_on_sc=False),
        ))
    return _SC_SCATTER_CACHE[0]


def _scatter(hL, hR, srcp, dstp, zrows):
    o2 = _get_sc_scatter()(hL, hR, srcp, dstp, zrows)
    return o2[0, :N], o2[1, :N]


def _split_body(ei_ref, src_ref, dst_ref):
    src_ref[pl.ds(0, E)] = ei_ref[0, :]
    dst_ref[pl.ds(0, E)] = ei_ref[1, :]
    src_ref[pl.ds(E, EPAD - E)] = jnp.zeros((EPAD - E,), jnp.int32)
    dst_ref[pl.ds(E, EPAD - E)] = jnp.full((EPAD - E,), SINK, jnp.int32)


def _tc_split_edges(edge_index):
    return pl.pallas_call(
        _split_body,
        out_shape=[
            jax.ShapeDtypeStruct((EPAD,), jnp.int32),
            jax.ShapeDtypeStruct((EPAD,), jnp.int32),
        ],
    )(edge_index.astype(jnp.int32))


def _emb_body(x_ref, emb_ref, xe_ref, l_ref, r_ref, z_ref, st_ref, acc_ref):
    i = pl.program_id(0)
    f32 = jnp.float32
    iota_e = lax.broadcasted_iota(jnp.int32, (ROWS, EMD), 1)
    acc = jnp.zeros((ROWS, H), f32)
    for k in range(FIN):
        oh = (x_ref[:, k][:, None] == iota_e).astype(f32)
        acc = acc + jnp.dot(oh, emb_ref[k], preferred_element_type=f32)
    xe_ref[...] = acc
    l_ref[...] = acc[:, :HW]
    r_ref[...] = acc[:, HW:]

    @pl.when(i == 0)
    def _():
        z_ref[...] = jnp.zeros_like(z_ref)
        acc_ref[...] = jnp.zeros_like(acc_ref)

    acc_ref[0, :] += jnp.sum(acc, axis=0)
    acc_ref[1, :] += jnp.sum(acc * acc, axis=0)

    @pl.when(i == NTILES - 1)
    def _():
        st_ref[...] = acc_ref[...]


def _tc_embed(x, atom_emb):
    return pl.pallas_call(
        _emb_body,
        grid=(NTILES,),
        in_specs=[
            pl.BlockSpec((ROWS, FIN), lambda i: (i, 0)),
            pl.BlockSpec((FIN, EMD, H), lambda i: (0, 0, 0)),
        ],
        out_specs=[
            pl.BlockSpec((ROWS, H), lambda i: (i, 0)),
            pl.BlockSpec((ROWS, HW), lambda i: (i, 0)),
            pl.BlockSpec((ROWS, HW), lambda i: (i, 0)),
            pl.BlockSpec((WR, HW), lambda i: (0, 0)),
            pl.BlockSpec((2, H), lambda i: (0, 0)),
        ],
        out_shape=[
            jax.ShapeDtypeStruct((N, H), jnp.float32),
            jax.ShapeDtypeStruct((N, HW), jnp.float32),
            jax.ShapeDtypeStruct((N, HW), jnp.float32),
            jax.ShapeDtypeStruct((WR, HW), jnp.float32),
            jax.ShapeDtypeStruct((2, H), jnp.float32),
        ],
        scratch_shapes=[pltpu.VMEM((2, H), jnp.float32)],
    )(x.astype(jnp.int32), atom_emb)


def _tail_body(e0_ref, e1_ref, e2_ref, b_ref, st_ref, Wg_ref, bg_ref, Wl_ref,
               bl_ref, Wc_ref, bc_ref, Wm1_ref, bm1_ref, Wm2_ref, bm2_ref,
               out_ref, kld_ref, diff_ref, mi_ref,
               segg_ref, segl_ref, M_ref, cjb_ref, ja0_ref, sm_ref):
    i = pl.program_id(0)
    f32 = jnp.float32
    dn = (((0,), (0,)), ((), ()))

    @pl.when(i == 0)
    def _():
        segg_ref[...] = jnp.zeros_like(segg_ref)
        segl_ref[...] = jnp.zeros_like(segl_ref)
        M_ref[...] = jnp.zeros_like(M_ref)
        sm_ref[0, 0] = -1e30
        sm_ref[0, 1] = 0.0
        sm_ref[0, 2] = 0.0

    e0 = e0_ref[...]
    e1 = e1_ref[...]
    e2 = e2_ref[...]
    bids = b_ref[0, 0, :]
    iota_g = lax.broadcasted_iota(jnp.int32, (ROWS, G), 1)
    oh = (bids[:, None] == iota_g).astype(f32)
    segg_ref[...] += lax.dot_general(oh, e1, dn, preferred_element_type=f32)
    segl_ref[...] += lax.dot_general(oh, e0, dn, preferred_element_type=f32)

    n0 = jnp.sqrt(jnp.sum(e0 * e0, axis=1, keepdims=True))
    n1 = jnp.sqrt(jnp.sum(e1 * e1, axis=1, keepdims=True))
    a2 = e0 / (n0 + 1e-6)
    b2 = e1 / (n1 + 1e-6)
    M_ref[...] += lax.dot_general(a2, b2, dn, preferred_element_type=f32)

    Wm1a = Wm1_ref[:H, :]
    Wm1b = Wm1_ref[H:, :]
    ja = jnp.dot(e1, Wm1a, preferred_element_type=f32)
    jb = jnp.dot(e2, Wm1b, preferred_element_type=f32)
    bm1 = bm1_ref[0, :]
    bm2 = bm2_ref[0, 0]
    joint_h = jnp.maximum(ja + jb + bm1, 0.0)
    joint_v = jnp.dot(joint_h, Wm2_ref[...], preferred_element_type=f32)[:, 0] + bm2
    sm_ref[0, 2] += jnp.sum(joint_v)

    jb_shift = jnp.concatenate([cjb_ref[...], jb[:ROWS - 1]], axis=0)
    cjb_ref[...] = jb[ROWS - 1:ROWS]

    @pl.when(i == 0)
    def _():
        ja0_ref[...] = ja[0:1]

    marg_h = jnp.maximum(ja + jb_shift + bm1, 0.0)
    marg_v = jnp.dot(marg_h, Wm2_ref[...], preferred_element_type=f32)[:, 0] + bm2
    rowid = lax.broadcasted_iota(jnp.int32, (ROWS,), 0)
    valid = (i > 0) | (rowid > 0)
    marg_v = jnp.where(valid, marg_v, -1e30)

    mold = sm_ref[0, 0]
    mnew = jnp.maximum(mold, jnp.max(marg_v))
    sm_ref[0, 1] = sm_ref[0, 1] * jnp.exp(mold - mnew) + jnp.sum(jnp.exp(marg_v - mnew))
    sm_ref[0, 0] = mnew

    @pl.when(i == NTILES - 1)
    def _():
        mh0 = jnp.maximum(ja0_ref[...] + cjb_ref[...] + bm1, 0.0)
        mv0 = jnp.dot(mh0, Wm2_ref[...], preferred_element_type=f32)[0, 0] + bm2
        m_all = jnp.maximum(sm_ref[0, 0], mv0)
        s_all = sm_ref[0, 1] * jnp.exp(sm_ref[0, 0] - m_all) + jnp.exp(mv0 - m_all)
        lse = jnp.log(s_all) + m_all
        mi_ref[...] = jnp.broadcast_to(sm_ref[0, 2] * (1.0 / N) - (lse - jnp.log(jnp.float32(N))), (1, 1))

        xg = jnp.maximum(jnp.dot(segg_ref[...], Wg_ref[...],
                                 preferred_element_type=f32) + bg_ref[0, :], 0.0)
        xl = jnp.maximum(jnp.dot(segl_ref[...], Wl_ref[...],
                                 preferred_element_type=f32) + bl_ref[0, :], 0.0)
        out_ref[...] = jnp.dot(xl + xg, Wc_ref[...],
                               preferred_element_type=f32) + bc_ref[0, :]

        Mv = M_ref[...]
        diff_ref[...] = jnp.broadcast_to(jnp.sum(Mv * Mv) * (1.0 / (H * H)), (1, 1))

        mu = st_ref[0, :] * (1.0 / N)
        var1 = (st_ref[1, :] - N * mu * mu) * (1.0 / (N - 1))
        kld_ref[...] = jnp.broadcast_to(-0.5 * jnp.mean(1.0 + jnp.log(var1) - mu * mu - var1), (1, 1))


def _tc_tail(e0, e1, e2, batch3, xstats, Wg, bg, Wl, bl, Wclf, bclf, Wm1, bm1, Wm2, bm2):
    cm = lambda i: (0, 0)
    return pl.pallas_call(
        _tail_body,
        grid=(NTILES,),
        in_specs=[
            pl.BlockSpec((ROWS, H), lambda i: (i, 0)),
            pl.BlockSpec((ROWS, H), lambda i: (i, 0)),
            pl.BlockSpec((ROWS, H), lambda i: (i, 0)),
            pl.BlockSpec((1, 1, ROWS), lambda i: (i, 0, 0)),
            pl.BlockSpec((2, H), cm),
            pl.BlockSpec((H, H), cm),
            pl.BlockSpec((1, H), cm),
            pl.BlockSpec((H, H), cm),
            pl.BlockSpec((1, H), cm),
            pl.BlockSpec((H, C), cm),
            pl.BlockSpec((1, C), cm),
            pl.BlockSpec((2 * H, H), cm),
            pl.BlockSpec((1, H), cm),
            pl.BlockSpec((H, 1), cm),
            pl.BlockSpec((1, 1), cm),
        ],
        out_specs=[
            pl.BlockSpec((G, C), cm),
            pl.BlockSpec((1, 1), cm),
            pl.BlockSpec((1, 1), cm),
            pl.BlockSpec((1, 1), cm),
        ],
        out_shape=[
            jax.ShapeDtypeStruct((G, C), jnp.float32),
            jax.ShapeDtypeStruct((1, 1), jnp.float32),
            jax.ShapeDtypeStruct((1, 1), jnp.float32),
            jax.ShapeDtypeStruct((1, 1), jnp.float32),
        ],
        scratch_shapes=[
            pltpu.VMEM((G, H), jnp.float32),
            pltpu.VMEM((G, H), jnp.float32),
            pltpu.VMEM((H, H), jnp.float32),
            pltpu.VMEM((1, H), jnp.float32),
            pltpu.VMEM((1, H), jnp.float32),
            pltpu.SMEM((1, 4), jnp.float32),
        ],
    )(e0, e1, e2, batch3, xstats, Wg, bg, Wl, bl, Wclf, bclf, Wm1, bm1, Wm2, bm2)


def _passA_gen_body(nh, *refs):
    i = pl.program_id(0)
    f32 = jnp.float32
    if nh == 1:
        h_ref, aL_ref, aR_ref, W_ref, b_ref, s_ref, stats_ref, acc_ref = refs
        ms = [h_ref[...] + jnp.concatenate([aL_ref[...], aR_ref[...]], axis=1)] * 3
    else:
        (h0, h1, h2, aL0, aR0, aL1, aR1, aL2, aR2,
         W_ref, b_ref, s_ref, stats_ref, acc_ref) = refs
        ms = [
            h0[...] + jnp.concatenate([aL0[...], aR0[...]], axis=1),
            h1[...] + jnp.concatenate([aL1[...], aR1[...]], axis=1),
            h2[...] + jnp.concatenate([aL2[...], aR2[...]], axis=1),
        ]

    @pl.when(i == 0)
    def _():
        acc_ref[...] = jnp.zeros_like(acc_ref)

    for k in range(3):
        s = jnp.dot(ms[k], W_ref[k], preferred_element_type=f32) + b_ref[k]
        s_ref[k] = s
        acc_ref[k, 0, :] += jnp.sum(s, axis=0)
        acc_ref[k, 1, :] += jnp.sum(s * s, axis=0)

    @pl.when(i == NTILES - 1)
    def _():
        stats_ref[...] = acc_ref[...]


def _passA(h, aggL, aggR, W1, b1):
    rb = lambda i: (i, 0)
    return pl.pallas_call(
        functools.partial(_passA_gen_body, 1),
        grid=(NTILES,),
        in_specs=[
            pl.BlockSpec((ROWS, H), rb),
            pl.BlockSpec((ROWS, HW), rb),
            pl.BlockSpec((ROWS, HW), rb),
            pl.BlockSpec((3, H, H), lambda i: (0, 0, 0)),
            pl.BlockSpec((3, 1, H), lambda i: (0, 0, 0)),
        ],
        out_specs=[
            pl.BlockSpec((3, ROWS, H), lambda i: (0, i, 0)),
            pl.BlockSpec((3, 2, H), lambda i: (0, 0, 0)),
        ],
        out_shape=[
            jax.ShapeDtypeStruct((3, N, H), jnp.float32),
            jax.ShapeDtypeStruct((3, 2, H), jnp.float32),
        ],
        scratch_shapes=[pltpu.VMEM((3, 2, H), jnp.float32)],
    )(h, aggL, aggR, W1, b1)


def _passA3(hs, aggs, W1, b1):
    rb = lambda i: (i, 0)
    h0, h1, h2 = hs
    (aL0, aR0), (aL1, aR1), (aL2, aR2) = aggs
    return pl.pallas_call(
        functools.partial(_passA_gen_body, 3),
        grid=(NTILES,),
        in_specs=[
            pl.BlockSpec((ROWS, H), rb),
            pl.BlockSpec((ROWS, H), rb),
            pl.BlockSpec((ROWS, H), rb),
        ] + [pl.BlockSpec((ROWS, HW), rb)] * 6 + [
            pl.BlockSpec((3, H, H), lambda i: (0, 0, 0)),
            pl.BlockSpec((3, 1, H), lambda i: (0, 0, 0)),
        ],
        out_specs=[
            pl.BlockSpec((3, ROWS, H), lambda i: (0, i, 0)),
            pl.BlockSpec((3, 2, H), lambda i: (0, 0, 0)),
        ],
        out_shape=[
            jax.ShapeDtypeStruct((3, N, H), jnp.float32),
            jax.ShapeDtypeStruct((3, 2, H), jnp.float32),
        ],
        scratch_shapes=[pltpu.VMEM((3, 2, H), jnp.float32)],
    )(h0, h1, h2, aL0, aR0, aL1, aR1, aL2, aR2, W1, b1)


def _passB3_body(relu_out, split, s_ref, stats_ref, gamma_ref, beta_ref, W2_ref, b2_ref,
                 *out_refs):
    f32 = jnp.float32
    for k in range(3):
        mu = stats_ref[k, 0, :] * (1.0 / N)
        var = stats_ref[k, 1, :] * (1.0 / N) - mu * mu
        inv = lax.rsqrt(var + 1e-5)
        m = (s_ref[k] - mu) * (inv * gamma_ref[k, 0, :]) + beta_ref[k, 0, :]
        m = jnp.maximum(m, 0.0)
        o = jnp.dot(m, W2_ref[k], preferred_element_type=f32) + b2_ref[k, 0, :]
        if relu_out:
            o = jnp.maximum(o, 0.0)
        out_refs[k][...] = o
        if split:
            out_refs[3 + 2 * k][...] = o[:, :HW]
            out_refs[4 + 2 * k][...] = o[:, HW:]


def _passB3(s3, stats3, gamma, beta, W2, b2, relu_out, split):
    rb = lambda i: (i, 0)
    cm = lambda i: (0, 0, 0)
    out_specs = [pl.BlockSpec((ROWS, H), rb)] * 3
    out_shape = [jax.ShapeDtypeStruct((N, H), jnp.float32)] * 3
    if split:
        out_specs += [pl.BlockSpec((ROWS, HW), rb)] * 6
        out_shape += [jax.ShapeDtypeStruct((N, HW), jnp.float32)] * 6
    return pl.pallas_call(
        functools.partial(_passB3_body, relu_out, split),
        grid=(NTILES,),
        in_specs=[
            pl.BlockSpec((3, ROWS, H), lambda i: (0, i, 0)),
            pl.BlockSpec((3, 2, H), cm),
            pl.BlockSpec((3, 1, H), cm),
            pl.BlockSpec((3, 1, H), cm),
            pl.BlockSpec((3, H, H), cm),
            pl.BlockSpec((3, 1, H), cm),
        ],
        out_specs=out_specs,
        out_shape=out_shape,
    )(s3, stats3, gamma, beta, W2, b2)


def kernel(x, edge_index, batch, atom_emb, gin_W1, gin_b1, gin_gamma, gin_beta,
           gin_W2, gin_b2, Wg, bg, Wl, bl, Wclf, bclf, Wm1, bm1, Wm2, bm2):
    f32 = jnp.float32
    srcp, dstp = _tc_split_edges(edge_index)
    xe, xeL, xeR, zrows, xstats = _tc_embed(x, atom_emb)

    b1r = gin_b1.reshape(3, DEPTH, 1, H)
    b2r = gin_b2.reshape(3, DEPTH, 1, H)
    gmr = gin_gamma.reshape(3, DEPTH, 1, H)
    btr = gin_beta.reshape(3, DEPTH, 1, H)

    aL0, aR0 = _scatter(xeL, xeR, srcp, dstp, zrows)
    s_all, stats_all = _passA(xe, aL0, aR0, gin_W1[:, 0], b1r[:, 0])
    (h0, h1, h2, hL0, hR0, hL1, hR1, hL2, hR2) = _passB3(
        s_all, stats_all, gmr[:, 0], btr[:, 0], gin_W2[:, 0], b2r[:, 0],
        relu_out=True, split=True)

    aggs = [_scatter(hL0, hR0, srcp, dstp, zrows),
            _scatter(hL1, hR1, srcp, dstp, zrows),
            _scatter(hL2, hR2, srcp, dstp, zrows)]
    s3, st3 = _passA3((h0, h1, h2), aggs, gin_W1[:, 1], b1r[:, 1])
    encs = _passB3(s3, st3, gmr[:, 1], btr[:, 1], gin_W2[:, 1], b2r[:, 1],
                   relu_out=False, split=False)

    batch3 = batch.astype(jnp.int32).reshape(NTILES, 1, ROWS)
    out, kld2, diff2, mi2 = _tc_tail(
        encs[0], encs[1], encs[2], batch3, xstats,
        Wg, bg.reshape(1, H), Wl, bl.reshape(1, H),
        Wclf, bclf.reshape(1, C), Wm1, bm1.reshape(1, H),
        Wm2, bm2.reshape(1, 1))
    return (out, kld2[0, 0], jnp.float32(0.0), diff2[0, 0], mi2[0, 0])
